# baseline (speedup 1.0000x reference)
import jax
import jax.numpy as jnp
from jax.experimental import pallas as pl


S_BLK = 256


def _add_kernel(x_ref, pe_ref, o_ref):
    o_ref[...] = x_ref[...] + pe_ref[...]


def kernel(x, pe_table):
    B, S, F = x.shape
    grid = (S // S_BLK,)
    return pl.pallas_call(
        _add_kernel,
        grid=grid,
        in_specs=[
            pl.BlockSpec((B, S_BLK, F), lambda i: (0, i, 0)),
            pl.BlockSpec((S_BLK, F), lambda i: (i, 0)),
        ],
        out_specs=pl.BlockSpec((B, S_BLK, F), lambda i: (0, i, 0)),
        out_shape=jax.ShapeDtypeStruct((B, S, F), x.dtype),
    )(x, pe_table)
